# baseline (speedup 1.0000x reference)
import functools
import jax
import jax.numpy as jnp
from jax import lax
from jax.experimental import pallas as pl
from jax.experimental.pallas import tpu as pltpu
from jax.experimental.pallas import tpu_sc as plsc

VOCAB = 100000
SEQ_LEN = 2048
D_MODEL = 768
BATCH = 4

NC, NS, L = 2, 16, 16
NW = NC * NS
S_PER_W = SEQ_LEN // NW
CHUNK = 16
NCHUNK = BATCH * S_PER_W // CHUNK
HPB = S_PER_W // CHUNK
DCH = D_MODEL // L


def _body(txt_hbm, seg_hbm, tok_hbm, pos_hbm, segtab_hbm, out_hbm,
          idx_all, sidx_all, pos_v, segtab_v, tok_v, res_v,
          sem_g0, sem_g1, sem_o0, sem_o1):
    wid = lax.axis_index("s") * NC + lax.axis_index("c")
    s0 = wid * S_PER_W
    gsems = [sem_g0, sem_g1]
    osems = [sem_o0, sem_o1]
    iota16 = lax.iota(jnp.int32, L)

    for b in range(BATCH):
        pltpu.sync_copy(txt_hbm.at[pl.ds(b * SEQ_LEN + s0, S_PER_W)],
                        idx_all.at[pl.ds(b * S_PER_W, S_PER_W)])
        pltpu.sync_copy(seg_hbm.at[pl.ds(b * SEQ_LEN + s0, S_PER_W)],
                        sidx_all.at[pl.ds(b * S_PER_W, S_PER_W)])
    pltpu.sync_copy(pos_hbm.at[pl.ds(s0, S_PER_W)], pos_v)
    pltpu.sync_copy(segtab_hbm, segtab_v)

    def gather_start(k):
        b, h = divmod(k, HPB)
        p = k % 2
        return pltpu.async_copy(
            tok_hbm.at[idx_all.at[pl.ds(b * S_PER_W + h * CHUNK, CHUNK)]],
            tok_v.at[p], gsems[p])

    cps = {0: gather_start(0)}
    outs = {}
    for k in range(NCHUNK):
        p = k % 2
        b, h = divmod(k, HPB)
        if k + 1 < NCHUNK:
            cps[k + 1] = gather_start(k + 1)
        cps[k].wait()
        if k >= 2:
            outs[k - 2].wait()

        def row(r, carry, p=p, b=b, h=h):
            seg_id = plsc.load_gather(
                sidx_all,
                [jnp.full((L,), b * S_PER_W + h * CHUNK, jnp.int32) + r])
            seg_base = seg_id * D_MODEL + iota16
            for j in range(DCH):
                sl = pl.ds(j * L, L)
                segvec = plsc.load_gather(segtab_v, [seg_base + (j * L)])
                res_v[p, r, sl] = (tok_v[p, r, sl]
                                   + pos_v[h * CHUNK + r, sl]
                                   + segvec)
            return carry

        lax.fori_loop(0, CHUNK, row, None)
        flat = b * SEQ_LEN + s0 + h * CHUNK
        outs[k] = pltpu.async_copy(res_v.at[p],
                                   out_hbm.at[pl.ds(flat, CHUNK)], osems[p])
    outs[NCHUNK - 2].wait()
    outs[NCHUNK - 1].wait()


@jax.jit
def _run(txt_flat, seg_flat, tok_table, pos_table, seg_table):
    mesh = plsc.VectorSubcoreMesh(core_axis_name="c", subcore_axis_name="s")
    k = functools.partial(
        pl.kernel,
        out_type=jax.ShapeDtypeStruct((BATCH * SEQ_LEN, D_MODEL), jnp.float32),
        mesh=mesh,
        compiler_params=pltpu.CompilerParams(needs_layout_passes=False),
        scratch_types=[
            pltpu.VMEM((BATCH * S_PER_W,), jnp.int32),
            pltpu.VMEM((BATCH * S_PER_W,), jnp.int32),
            pltpu.VMEM((S_PER_W, D_MODEL), jnp.float32),
            pltpu.VMEM((3 * D_MODEL,), jnp.float32),
            pltpu.VMEM((2, CHUNK, D_MODEL), jnp.float32),
            pltpu.VMEM((2, CHUNK, D_MODEL), jnp.float32),
            pltpu.SemaphoreType.DMA,
            pltpu.SemaphoreType.DMA,
            pltpu.SemaphoreType.DMA,
            pltpu.SemaphoreType.DMA,
        ],
    )(_body)
    return k(txt_flat, seg_flat, tok_table, pos_table, seg_table)


def kernel(txt, seg, tok_table, pos_table, seg_table):
    txt_flat = txt.reshape(-1).astype(jnp.int32)
    seg_flat = seg.reshape(-1).astype(jnp.int32)
    out = _run(txt_flat, seg_flat, tok_table, pos_table,
               seg_table.reshape(-1))
    return out.reshape(BATCH, SEQ_LEN, D_MODEL)


# speedup vs baseline: 1.5877x; 1.5877x over previous
import functools
import jax
import jax.numpy as jnp
from jax import lax
from jax.experimental import pallas as pl
from jax.experimental.pallas import tpu as pltpu
from jax.experimental.pallas import tpu_sc as plsc

VOCAB = 100000
SEQ_LEN = 2048
D_MODEL = 768
BATCH = 4

NC, NS, L = 2, 16, 16
NW = NC * NS
S_PER_W = SEQ_LEN // NW
CHUNK = 32
NCHUNK = BATCH * S_PER_W // CHUNK
HPB = S_PER_W // CHUNK
DCH = D_MODEL // L
GRP = 8


def _body(txt_hbm, seg_hbm, tok_hbm, pos_hbm, segtab_hbm, out_hbm,
          idx_all, sidx_all, pos_v, segtab_v, res_v,
          sem_g0, sem_g1, sem_o0, sem_o1):
    wid = lax.axis_index("s") * NC + lax.axis_index("c")
    s0 = wid * S_PER_W
    gsems = [sem_g0, sem_g1]
    osems = [sem_o0, sem_o1]

    for b in range(BATCH):
        pltpu.sync_copy(txt_hbm.at[pl.ds(b * SEQ_LEN + s0, S_PER_W)],
                        idx_all.at[pl.ds(b * S_PER_W, S_PER_W)])
        pltpu.sync_copy(seg_hbm.at[pl.ds(b * SEQ_LEN + s0, S_PER_W)],
                        sidx_all.at[pl.ds(b * S_PER_W, S_PER_W)])
    pltpu.sync_copy(pos_hbm.at[pl.ds(s0, S_PER_W)], pos_v)
    pltpu.sync_copy(segtab_hbm, segtab_v)

    def gather_start(k):
        b, h = divmod(k, HPB)
        p = k % 2
        return pltpu.async_copy(
            tok_hbm.at[idx_all.at[pl.ds(b * S_PER_W + h * CHUNK, CHUNK)]],
            res_v.at[p], gsems[p])

    cps = {0: gather_start(0)}
    outs = {}
    for k in range(NCHUNK):
        p = k % 2
        b, h = divmod(k, HPB)
        if k + 1 < NCHUNK:
            if k >= 1:
                outs[k - 1].wait()
            cps[k + 1] = gather_start(k + 1)
        cps[k].wait()

        def row(r, carry, p=p, b=b, h=h):
            segv = sidx_all[pl.ds(b * S_PER_W + h * CHUNK + r, L)]
            seg_off = segv[0] * D_MODEL

            def loads(g):
                ts = []
                for j in range(g * GRP, (g + 1) * GRP):
                    ts.append(segtab_v[pl.ds(seg_off + j * L, L)]
                              + pos_v[h * CHUNK + r, pl.ds(j * L, L)])
                return ts

            def stores(g, ts):
                for i, j in enumerate(range(g * GRP, (g + 1) * GRP)):
                    plsc.addupdate(res_v.at[p, r, pl.ds(j * L, L)], ts[i])

            ngrp = DCH // GRP
            vals = loads(0)
            for g in range(ngrp):
                nxt = loads(g + 1) if g + 1 < ngrp else None
                stores(g, vals)
                vals = nxt
            return carry

        lax.fori_loop(0, CHUNK, row, None)
        flat = b * SEQ_LEN + s0 + h * CHUNK
        outs[k] = pltpu.async_copy(res_v.at[p],
                                   out_hbm.at[pl.ds(flat, CHUNK)], osems[p])
    outs[NCHUNK - 2].wait()
    outs[NCHUNK - 1].wait()


@jax.jit
def _run(txt_flat, seg_flat, tok_table, pos_table, seg_table):
    mesh = plsc.VectorSubcoreMesh(core_axis_name="c", subcore_axis_name="s")
    k = functools.partial(
        pl.kernel,
        out_type=jax.ShapeDtypeStruct((BATCH * SEQ_LEN, D_MODEL), jnp.float32),
        mesh=mesh,
        compiler_params=pltpu.CompilerParams(needs_layout_passes=False),
        scratch_types=[
            pltpu.VMEM((BATCH * S_PER_W,), jnp.int32),
            pltpu.VMEM((BATCH * S_PER_W + L,), jnp.int32),
            pltpu.VMEM((S_PER_W, D_MODEL), jnp.float32),
            pltpu.VMEM((3 * D_MODEL,), jnp.float32),
            pltpu.VMEM((2, CHUNK, D_MODEL), jnp.float32),
            pltpu.SemaphoreType.DMA,
            pltpu.SemaphoreType.DMA,
            pltpu.SemaphoreType.DMA,
            pltpu.SemaphoreType.DMA,
        ],
    )(_body)
    return k(txt_flat, seg_flat, tok_table, pos_table, seg_table)


def kernel(txt, seg, tok_table, pos_table, seg_table):
    txt_flat = txt.reshape(-1).astype(jnp.int32)
    seg_flat = seg.reshape(-1).astype(jnp.int32)
    out = _run(txt_flat, seg_flat, tok_table, pos_table,
               seg_table.reshape(-1))
    return out.reshape(BATCH, SEQ_LEN, D_MODEL)


# speedup vs baseline: 1.6695x; 1.0515x over previous
import functools
import jax
import jax.numpy as jnp
from jax import lax
from jax.experimental import pallas as pl
from jax.experimental.pallas import tpu as pltpu
from jax.experimental.pallas import tpu_sc as plsc

VOCAB = 100000
SEQ_LEN = 2048
D_MODEL = 768
BATCH = 4

NC, NS, L = 2, 16, 16
NW = NC * NS
S_PER_W = SEQ_LEN // NW
CHUNK = 32
NCHUNK = BATCH * S_PER_W // CHUNK
HPB = S_PER_W // CHUNK
DCH = D_MODEL // L
GRP = 8


def _body(txt_hbm, seg_hbm, tok_hbm, pos_hbm, segtab_hbm, out_hbm,
          idx_all, sidx_all, pos_v, segtab_v, res_v,
          sem_g0, sem_g1, sem_o0, sem_o1):
    wid = lax.axis_index("s") * NC + lax.axis_index("c")
    s0 = wid * S_PER_W
    gsems = [sem_g0, sem_g1]
    osems = [sem_o0, sem_o1]

    for b in range(BATCH):
        pltpu.sync_copy(txt_hbm.at[pl.ds(b * SEQ_LEN + s0, S_PER_W)],
                        idx_all.at[pl.ds(b * S_PER_W, S_PER_W)])
        pltpu.sync_copy(seg_hbm.at[pl.ds(b * SEQ_LEN + s0, S_PER_W)],
                        sidx_all.at[pl.ds(b * S_PER_W, S_PER_W)])
    pltpu.sync_copy(pos_hbm.at[pl.ds(s0, S_PER_W)], pos_v)
    pltpu.sync_copy(segtab_hbm, segtab_v)

    def gather_start(k):
        b, h = divmod(k, HPB)
        p = k % 2
        return pltpu.async_copy(
            tok_hbm.at[idx_all.at[pl.ds(b * S_PER_W + h * CHUNK, CHUNK)]],
            res_v.at[p], gsems[p])

    cps = {0: gather_start(0)}
    outs = {}
    for k in range(NCHUNK):
        p = k % 2
        b, h = divmod(k, HPB)
        if k + 1 < NCHUNK:
            if k >= 1:
                outs[k - 1].wait()
            cps[k + 1] = gather_start(k + 1)
        cps[k].wait()

        def seg_off_of(r, b=b, h=h):
            segv = sidx_all[pl.ds(b * S_PER_W + h * CHUNK + r, L)]
            return segv[0] * D_MODEL

        def row(r, seg_off, p=p, b=b, h=h):
            nxt_off = seg_off_of(r + 1)

            def loads(g):
                ts = []
                for j in range(g * GRP, (g + 1) * GRP):
                    ts.append(segtab_v[pl.ds(seg_off + j * L, L)]
                              + pos_v[h * CHUNK + r, pl.ds(j * L, L)])
                return ts

            def stores(g, ts):
                for i, j in enumerate(range(g * GRP, (g + 1) * GRP)):
                    plsc.addupdate(res_v.at[p, r, pl.ds(j * L, L)], ts[i])

            ngrp = DCH // GRP
            vals = loads(0)
            for g in range(ngrp):
                nxt = loads(g + 1) if g + 1 < ngrp else None
                stores(g, vals)
                vals = nxt
            return nxt_off

        lax.fori_loop(0, CHUNK, row, seg_off_of(0))
        flat = b * SEQ_LEN + s0 + h * CHUNK
        outs[k] = pltpu.async_copy(res_v.at[p],
                                   out_hbm.at[pl.ds(flat, CHUNK)], osems[p])
    outs[NCHUNK - 2].wait()
    outs[NCHUNK - 1].wait()


@jax.jit
def _run(txt_flat, seg_flat, tok_table, pos_table, seg_table):
    mesh = plsc.VectorSubcoreMesh(core_axis_name="c", subcore_axis_name="s")
    k = functools.partial(
        pl.kernel,
        out_type=jax.ShapeDtypeStruct((BATCH * SEQ_LEN, D_MODEL), jnp.float32),
        mesh=mesh,
        compiler_params=pltpu.CompilerParams(needs_layout_passes=False),
        scratch_types=[
            pltpu.VMEM((BATCH * S_PER_W,), jnp.int32),
            pltpu.VMEM((BATCH * S_PER_W + L,), jnp.int32),
            pltpu.VMEM((S_PER_W, D_MODEL), jnp.float32),
            pltpu.VMEM((3 * D_MODEL,), jnp.float32),
            pltpu.VMEM((2, CHUNK, D_MODEL), jnp.float32),
            pltpu.SemaphoreType.DMA,
            pltpu.SemaphoreType.DMA,
            pltpu.SemaphoreType.DMA,
            pltpu.SemaphoreType.DMA,
        ],
    )(_body)
    return k(txt_flat, seg_flat, tok_table, pos_table, seg_table)


def kernel(txt, seg, tok_table, pos_table, seg_table):
    txt_flat = txt.reshape(-1).astype(jnp.int32)
    seg_flat = seg.reshape(-1).astype(jnp.int32)
    out = _run(txt_flat, seg_flat, tok_table, pos_table,
               seg_table.reshape(-1))
    return out.reshape(BATCH, SEQ_LEN, D_MODEL)
